# Initial kernel scaffold; baseline (speedup 1.0000x reference)
import jax
import jax.numpy as jnp
from jax.experimental import pallas as pl


def kernel(x, perm):
    raise NotImplementedError("write your pallas kernel here")


# speedup vs baseline: 22.4340x; 22.4340x over previous
import functools

import jax
import jax.numpy as jnp
from jax.experimental import pallas as pl
from jax.experimental.pallas import tpu as pltpu

_B, _N, _C = 8, 576, 384
_K = _N // 4
_BN = _B * _N
_ITERS = 10


def _dot(a, b, contract, prec):
    return jax.lax.dot_general(
        a, b, (contract, ((), ())),
        precision=prec,
        preferred_element_type=jnp.float32)


def _kmeans_body(x_ref, perm_ref, out_ref):
    hi = jax.lax.Precision.HIGHEST
    de = jax.lax.Precision.DEFAULT
    x = x_ref[...]
    pid = perm_ref[...]

    init_oh = (pid == jax.lax.broadcasted_iota(
        jnp.int32, (_K, _BN), 1)).astype(jnp.float32)
    centroids = _dot(init_oh, x, ((1,), (0,)), hi)

    a2 = jnp.sum(x * x, axis=1, keepdims=True)

    def assign(centroids):
        b2 = jnp.sum(centroids * centroids, axis=1)
        xc = _dot(x, centroids, ((1,), (1,)), de)
        d2 = jnp.maximum(a2 + b2[None, :] - 2.0 * xc, 0.0)
        dd = jnp.sqrt(d2)
        return jnp.argmin(dd, axis=1).astype(jnp.int32)[:, None]

    def iter_body(_, centroids):
        ci = assign(centroids)
        oh = (ci == jax.lax.broadcasted_iota(
            jnp.int32, (_BN, _K), 1)).astype(jnp.float32)
        sums = _dot(oh, x, ((0,), (0,)), hi)
        counts = jnp.sum(oh, axis=0)
        return jnp.where(counts[:, None] > 0,
                         sums / jnp.maximum(counts, 1.0)[:, None],
                         jnp.zeros((_K, _C), jnp.float32))

    centroids = jax.lax.fori_loop(0, _ITERS - 1, iter_body, centroids,
                                  unroll=False)
    ci = assign(centroids)
    oh = (ci == jax.lax.broadcasted_iota(
        jnp.int32, (_BN, _K), 1)).astype(jnp.float32)

    for b in range(_B):
        oh_b = oh[b * _N:(b + 1) * _N]
        x_b = x[b * _N:(b + 1) * _N]
        sums_b = _dot(oh_b, x_b, ((0,), (0,)), hi)
        counts_b = jnp.sum(oh_b, axis=0)
        out_ref[b, :, :] = jnp.where(
            counts_b[:, None] > 0,
            sums_b / jnp.maximum(counts_b, 1.0)[:, None],
            jnp.zeros((_K, _C), jnp.float32))


@functools.partial(jax.jit, static_argnames=())
def kernel(x, perm):
    x_flat = x.reshape(_BN, _C)
    perm144 = perm[:_K].astype(jnp.int32).reshape(_K, 1)
    out = pl.pallas_call(
        _kmeans_body,
        out_shape=jax.ShapeDtypeStruct((_B, _K, _C), jnp.float32),
        compiler_params=pltpu.CompilerParams(
            vmem_limit_bytes=96 * 1024 * 1024),
    )(x_flat, perm144)
    return out
